# Initial kernel scaffold; baseline (speedup 1.0000x reference)
import jax
import jax.numpy as jnp
from jax.experimental import pallas as pl


def kernel(values, gather_idx, segment_ids):
    raise NotImplementedError("write your pallas kernel here")


# speedup vs baseline: 2.3124x; 2.3124x over previous
import jax
import jax.numpy as jnp
from jax import lax
from jax.experimental import pallas as pl
from jax.experimental.pallas import tpu as pltpu
from jax.experimental.pallas import tpu_sc as plsc

N_SRC = 10000
E = 160000
D = 256
N_SEG = 10000

NC = 2
NS = 16
L = 16

DH = D // 2
SPW = 320
TRASH = SPW
ACC_ROWS = SPW + 8

CH = 128
NBLK = E // 8


def _sc_body(values_hbm, gidx_hbm, seg_hbm, out_hbm,
             rows_v, acc_v, cnt_v, segc_v, gidx_c, sbuf, sem):
    c = lax.axis_index("c")
    s = lax.axis_index("s")
    w = c * NS + s
    g0 = w * SPW
    g1 = g0 + SPW

    def lower_bound8(target):
        lo = jnp.int32(0)
        step = 1 << 14
        while step >= 1:
            cand = lo + step
            candc = jnp.minimum(cand, NBLK)
            pltpu.sync_copy(seg_hbm.at[pl.ds((candc - 1) * 8, L)], sbuf)
            val = sbuf[pl.ds(0, L)][0]
            ok = jnp.logical_and(cand <= NBLK, val < target)
            lo = jnp.where(ok, cand, lo)
            step //= 2
        return lo

    m_lo = lower_bound8(g0)
    m_hi = lower_bound8(g1)
    e_start = 8 * jnp.maximum(m_lo - 1, 0)
    e_end = 8 * m_hi
    nch = (e_end - e_start + CH - 1) // CH

    zeros = jnp.zeros((L,), jnp.float32)
    onehot0 = jnp.where(lax.iota(jnp.int32, L) == 0, 1.0, 0.0)

    for h in range(2):
        def zero_row(r, _):
            for j in range(DH // L):
                acc_v[r, pl.ds(j * L, L)] = zeros
            if h == 0:
                cnt_v[r, :] = zeros
            return 0

        lax.fori_loop(0, SPW + 1, zero_row, 0)

        def chunk(k, _):
            off = e_start + k * CH
            pltpu.sync_copy(seg_hbm.at[pl.ds(off, CH)], segc_v)
            pltpu.sync_copy(gidx_hbm.at[pl.ds(off, CH)], gidx_c)
            for i in range(CH // L):
                gv = gidx_c[pl.ds(i * L, L)]
                gidx_c[pl.ds(i * L, L)] = gv * 2 + h
            pltpu.async_copy(values_hbm.at[gidx_c], rows_v, sem).wait()

            def group(i, _):
                sv = segc_v[pl.ds(i * L, L)]
                valid = jnp.logical_and(sv >= g0, sv < g1)
                lv = jnp.where(valid, sv - g0, TRASH)
                for t in range(L):
                    le = lv[t]
                    e = i * L + t
                    for j in range(DH // L):
                        acc_v[le, pl.ds(j * L, L)] = (
                            acc_v[le, pl.ds(j * L, L)]
                            + rows_v[e, pl.ds(j * L, L)])
                    if h == 0:
                        cnt_v[le, :] = cnt_v[le, :] + onehot0
                return 0

            lax.fori_loop(0, CH // L, group, 0)
            return 0

        lax.fori_loop(0, nch, chunk, 0)

        def div_row(r, _):
            cw = cnt_v[r, :]
            recipv = 1.0 / jnp.maximum(cw, 1.0)
            recip = jnp.full((L,), recipv[0])
            for j in range(DH // L):
                acc_v[r, pl.ds(j * L, L)] = acc_v[r, pl.ds(j * L, L)] * recip
            return 0

        lax.fori_loop(0, SPW, div_row, 0)

        pltpu.sync_copy(acc_v.at[pl.ds(0, SPW)],
                        out_hbm.at[h].at[pl.ds(w * SPW, SPW)])


@jax.jit
def _sc_call(values, gather_idx, segment_ids):
    mesh = plsc.VectorSubcoreMesh(core_axis_name="c", subcore_axis_name="s")
    kfn = pl.kernel(
        _sc_body,
        mesh=mesh,
        out_type=jax.ShapeDtypeStruct((2, NC * NS * SPW, DH), jnp.float32),
        scratch_types=[
            pltpu.VMEM((CH, DH), jnp.float32),
            pltpu.VMEM((ACC_ROWS, DH), jnp.float32),
            pltpu.VMEM((ACC_ROWS, L), jnp.float32),
            pltpu.VMEM((CH,), jnp.int32),
            pltpu.VMEM((CH,), jnp.int32),
            pltpu.VMEM((L,), jnp.int32),
            pltpu.SemaphoreType.DMA,
        ],
    )
    vh = values.reshape(2 * N_SRC, DH)
    gi = jnp.concatenate([gather_idx, jnp.zeros((CH,), jnp.int32)])
    si = jnp.concatenate(
        [segment_ids, jnp.full((CH,), jnp.int32(2**30 - 1), jnp.int32)])
    return kfn(vh, gi, si)


def kernel(values, gather_idx, segment_ids):
    o = _sc_call(values, gather_idx, segment_ids)
    return jnp.concatenate([o[0, :N_SEG], o[1, :N_SEG]], axis=1)


# speedup vs baseline: 2.9445x; 1.2734x over previous
import jax
import jax.numpy as jnp
from jax import lax
from jax.experimental import pallas as pl
from jax.experimental.pallas import tpu as pltpu
from jax.experimental.pallas import tpu_sc as plsc

N_SRC = 10000
E = 160000
D = 256
N_SEG = 10000

NC = 2
NS = 16
L = 16

DH = D // 2
SPW = 320
TRASH = SPW
ACC_ROWS = SPW + 8

CH = 128
PAD = 8 * CH
NBLK = E // 8


def _sc_body(values_hbm, gidx_hbm, seg_hbm, out_hbm,
             rows_a, rows_b, acc_v, cnt_v,
             segc_0, segc_1, segc_2, segc_3,
             gidx_0, gidx_1, gidx_2, gidx_3,
             sbuf,
             gsem_0, gsem_1, ssem_0, ssem_1, ssem_2, ssem_3):
    rows = [rows_a, rows_b]
    segc = [segc_0, segc_1, segc_2, segc_3]
    gidx = [gidx_0, gidx_1, gidx_2, gidx_3]
    gsem = [gsem_0, gsem_1]
    ssem = [ssem_0, ssem_1, ssem_2, ssem_3]

    c = lax.axis_index("c")
    s = lax.axis_index("s")
    w = c * NS + s
    g0 = w * SPW
    g1 = g0 + SPW

    def lower_bound8(target):
        lo = jnp.int32(0)
        step = 1 << 14
        while step >= 1:
            cand = lo + step
            candc = jnp.minimum(cand, NBLK)
            pltpu.sync_copy(seg_hbm.at[pl.ds((candc - 1) * 8, L)], sbuf)
            val = sbuf[pl.ds(0, L)][0]
            ok = jnp.logical_and(cand <= NBLK, val < target)
            lo = jnp.where(ok, cand, lo)
            step //= 2
        return lo

    m_lo = lower_bound8(g0)
    m_hi = lower_bound8(g1)
    e_start = 8 * jnp.maximum(m_lo - 1, 0)
    e_end = 8 * m_hi
    nch = (e_end - e_start + CH - 1) // CH
    nq = jnp.maximum((nch + 3) // 4, 1)

    zeros = jnp.zeros((L,), jnp.float32)
    onehot0 = jnp.where(lax.iota(jnp.int32, L) == 0, 1.0, 0.0)

    for h in range(2):
        def zero_row(r, _):
            for j in range(DH // L):
                acc_v[r, pl.ds(j * L, L)] = zeros
            if h == 0:
                cnt_v[r, :] = zeros
            return 0

        lax.fori_loop(0, SPW + 1, zero_row, 0)

        def small_issue(koff, slot):
            off = e_start + koff * CH
            pltpu.async_copy(seg_hbm.at[pl.ds(off, CH)], segc[slot],
                             ssem[slot])
            pltpu.async_copy(gidx_hbm.at[pl.ds(off, CH)], gidx[slot],
                             ssem[slot])

        def small_wait(slot):
            pltpu.make_async_copy(seg_hbm.at[pl.ds(0, CH)], segc[slot],
                                  ssem[slot]).wait()
            pltpu.make_async_copy(gidx_hbm.at[pl.ds(0, CH)], gidx[slot],
                                  ssem[slot]).wait()

        def transform(slot):
            for i in range(CH // L):
                gv = gidx[slot][pl.ds(i * L, L)]
                gidx[slot][pl.ds(i * L, L)] = gv * 2 + h

        def gather_issue(slot4, slot2):
            pltpu.async_copy(values_hbm.at[gidx[slot4]], rows[slot2],
                             gsem[slot2])

        def gather_wait(slot2):
            pltpu.make_async_copy(values_hbm.at[pl.ds(0, CH)], rows[slot2],
                                  gsem[slot2]).wait()

        def process(segc_ref, rows_ref):
            def group(i, _):
                sv = segc_ref[pl.ds(i * L, L)]
                valid = jnp.logical_and(sv >= g0, sv < g1)
                lv = jnp.where(valid, sv - g0, TRASH)
                for t in range(L):
                    le = lv[t]
                    e = i * L + t
                    for j in range(DH // L):
                        acc_v[le, pl.ds(j * L, L)] = (
                            acc_v[le, pl.ds(j * L, L)]
                            + rows_ref[e, pl.ds(j * L, L)])
                    if h == 0:
                        cnt_v[le, :] = cnt_v[le, :] + onehot0
                return 0

            lax.fori_loop(0, CH // L, group, 0)

        pltpu.sync_copy(seg_hbm.at[pl.ds(e_start, CH)], segc[0])
        pltpu.sync_copy(gidx_hbm.at[pl.ds(e_start, CH)], gidx[0])
        transform(0)
        gather_issue(0, 0)
        small_issue(1, 1)

        def quad(q, _):
            k_base = q * 4
            for b in range(4):
                k = k_base + b
                small_wait((b + 1) % 4)
                transform((b + 1) % 4)
                gather_issue((b + 1) % 4, (b + 1) % 2)
                small_issue(k + 2, (b + 2) % 4)
                gather_wait(b % 2)
                process(segc[b], rows[b % 2])
            return 0

        lax.fori_loop(0, nq, quad, 0)

        small_wait(1)
        gather_wait(0)

        def div_row(r, _):
            cw = cnt_v[r, :]
            recipv = 1.0 / jnp.maximum(cw, 1.0)
            recip = jnp.full((L,), recipv[0])
            for j in range(DH // L):
                acc_v[r, pl.ds(j * L, L)] = acc_v[r, pl.ds(j * L, L)] * recip
            return 0

        lax.fori_loop(0, SPW, div_row, 0)

        pltpu.sync_copy(acc_v.at[pl.ds(0, SPW)],
                        out_hbm.at[h].at[pl.ds(w * SPW, SPW)])


@jax.jit
def _sc_call(values, gather_idx, segment_ids):
    mesh = plsc.VectorSubcoreMesh(core_axis_name="c", subcore_axis_name="s")
    kfn = pl.kernel(
        _sc_body,
        mesh=mesh,
        out_type=jax.ShapeDtypeStruct((2, NC * NS * SPW, DH), jnp.float32),
        scratch_types=[
            pltpu.VMEM((CH, DH), jnp.float32),
            pltpu.VMEM((CH, DH), jnp.float32),
            pltpu.VMEM((ACC_ROWS, DH), jnp.float32),
            pltpu.VMEM((ACC_ROWS, L), jnp.float32),
            pltpu.VMEM((CH,), jnp.int32),
            pltpu.VMEM((CH,), jnp.int32),
            pltpu.VMEM((CH,), jnp.int32),
            pltpu.VMEM((CH,), jnp.int32),
            pltpu.VMEM((CH,), jnp.int32),
            pltpu.VMEM((CH,), jnp.int32),
            pltpu.VMEM((CH,), jnp.int32),
            pltpu.VMEM((CH,), jnp.int32),
            pltpu.VMEM((L,), jnp.int32),
            pltpu.SemaphoreType.DMA,
            pltpu.SemaphoreType.DMA,
            pltpu.SemaphoreType.DMA,
            pltpu.SemaphoreType.DMA,
            pltpu.SemaphoreType.DMA,
            pltpu.SemaphoreType.DMA,
        ],
    )
    vh = values.reshape(2 * N_SRC, DH)
    gi = jnp.concatenate([gather_idx, jnp.zeros((PAD,), jnp.int32)])
    si = jnp.concatenate(
        [segment_ids, jnp.full((PAD,), jnp.int32(2**30 - 1), jnp.int32)])
    return kfn(vh, gi, si)


def kernel(values, gather_idx, segment_ids):
    o = _sc_call(values, gather_idx, segment_ids)
    return jnp.concatenate([o[0, :N_SEG], o[1, :N_SEG]], axis=1)


# speedup vs baseline: 5.7141x; 1.9406x over previous
import jax
import jax.numpy as jnp
from jax import lax
from jax.experimental import pallas as pl
from jax.experimental.pallas import tpu as pltpu
from jax.experimental.pallas import tpu_sc as plsc

N_SRC = 10000
E = 160000
D = 256
N_SEG = 10000

NC = 2
NS = 16
L = 16

DH = D // 2
SPW = 320
TRASH = SPW
ACC_ROWS = SPW + 8

CH = 128
PAD = 8 * CH
NBLK = E // 8


def _sc_body(values_hbm, gidx_hbm, seg_hbm, out_hbm,
             rows_a, rows_b, acc_v, cnt_v,
             segc_0, segc_1, segc_2, segc_3,
             gidx_0, gidx_1, gidx_2, gidx_3,
             sbuf,
             gsem_0, gsem_1, ssem_0, ssem_1, ssem_2, ssem_3):
    rows = [rows_a, rows_b]
    segc = [segc_0, segc_1, segc_2, segc_3]
    gidx = [gidx_0, gidx_1, gidx_2, gidx_3]
    gsem = [gsem_0, gsem_1]
    ssem = [ssem_0, ssem_1, ssem_2, ssem_3]

    c = lax.axis_index("c")
    s = lax.axis_index("s")
    w = c * NS + s
    g0 = w * SPW
    g1 = g0 + SPW

    def lower_bound8(target):
        lo = jnp.int32(0)
        step = 1 << 14
        while step >= 1:
            cand = lo + step
            candc = jnp.minimum(cand, NBLK)
            pltpu.sync_copy(seg_hbm.at[pl.ds((candc - 1) * 8, L)], sbuf)
            val = sbuf[pl.ds(0, L)][0]
            ok = jnp.logical_and(cand <= NBLK, val < target)
            lo = jnp.where(ok, cand, lo)
            step //= 2
        return lo

    m_lo = lower_bound8(g0)
    m_hi = lower_bound8(g1)
    e_start = 8 * jnp.maximum(m_lo - 1, 0)
    e_end = 8 * m_hi
    nch = (e_end - e_start + CH - 1) // CH
    nq = jnp.maximum((nch + 3) // 4, 1)

    zeros = jnp.zeros((L,), jnp.float32)
    onehot0 = jnp.where(lax.iota(jnp.int32, L) == 0, 1.0, 0.0)

    for h in range(2):
        def zero_row(r, _):
            for j in range(DH // L):
                acc_v[r, pl.ds(j * L, L)] = zeros
            if h == 0:
                cnt_v[r, :] = zeros
            return 0

        lax.fori_loop(0, SPW + 1, zero_row, 0)

        def small_issue(koff, slot):
            off = e_start + koff * CH
            pltpu.async_copy(seg_hbm.at[pl.ds(off, CH)], segc[slot],
                             ssem[slot])
            pltpu.async_copy(gidx_hbm.at[pl.ds(off, CH)], gidx[slot],
                             ssem[slot])

        def small_wait(slot):
            pltpu.make_async_copy(seg_hbm.at[pl.ds(0, CH)], segc[slot],
                                  ssem[slot]).wait()
            pltpu.make_async_copy(gidx_hbm.at[pl.ds(0, CH)], gidx[slot],
                                  ssem[slot]).wait()

        def transform(slot):
            for i in range(CH // L):
                gv = gidx[slot][pl.ds(i * L, L)]
                gidx[slot][pl.ds(i * L, L)] = gv * 2 + h

        def gather_issue(slot4, slot2):
            pltpu.async_copy(values_hbm.at[gidx[slot4]], rows[slot2],
                             gsem[slot2])

        def gather_wait(slot2):
            pltpu.make_async_copy(values_hbm.at[pl.ds(0, CH)], rows[slot2],
                                  gsem[slot2]).wait()

        def flush(carry):
            prev, rcnt, regs = carry
            for j in range(DH // L):
                acc_v[prev, pl.ds(j * L, L)] = (
                    acc_v[prev, pl.ds(j * L, L)] + regs[j])
            if h == 0:
                cntf = jnp.full((L,), rcnt.astype(jnp.float32))
                cnt_v[prev, :] = cnt_v[prev, :] + onehot0 * cntf

        def process(segc_ref, rows_ref, carry):
            def group(i, carry):
                sv = segc_ref[pl.ds(i * L, L)]
                valid = jnp.logical_and(sv >= g0, sv < g1)
                lv = jnp.where(valid, sv - g0, TRASH)
                for t in range(L):
                    le = lv[t]
                    e = i * L + t
                    row = [rows_ref[e, pl.ds(j * L, L)]
                           for j in range(DH // L)]
                    prev, rcnt, regs = carry

                    is_new = le != prev
                    lax.cond(is_new, lambda: flush(carry), lambda: None)
                    regs = [jnp.where(is_new, row[j], regs[j] + row[j])
                            for j in range(DH // L)]
                    rcnt = jnp.where(is_new, jnp.int32(1), rcnt + 1)
                    carry = (le, rcnt, regs)
                return carry

            return lax.fori_loop(0, CH // L, group, carry)

        pltpu.sync_copy(seg_hbm.at[pl.ds(e_start, CH)], segc[0])
        pltpu.sync_copy(gidx_hbm.at[pl.ds(e_start, CH)], gidx[0])
        transform(0)
        gather_issue(0, 0)
        small_issue(1, 1)

        def quad(q, carry):
            k_base = q * 4
            for b in range(4):
                k = k_base + b
                small_wait((b + 1) % 4)
                transform((b + 1) % 4)
                gather_issue((b + 1) % 4, (b + 1) % 2)
                small_issue(k + 2, (b + 2) % 4)
                gather_wait(b % 2)
                carry = process(segc[b], rows[b % 2], carry)
            return carry

        carry0 = (jnp.int32(TRASH), jnp.int32(0),
                  [zeros for _ in range(DH // L)])
        carry = lax.fori_loop(0, nq, quad, carry0)
        flush(carry)

        small_wait(1)
        gather_wait(0)

        def div_row(r, _):
            cw = cnt_v[r, :]
            recipv = 1.0 / jnp.maximum(cw, 1.0)
            recip = jnp.full((L,), recipv[0])
            for j in range(DH // L):
                acc_v[r, pl.ds(j * L, L)] = acc_v[r, pl.ds(j * L, L)] * recip
            return 0

        lax.fori_loop(0, SPW, div_row, 0)

        pltpu.sync_copy(acc_v.at[pl.ds(0, SPW)],
                        out_hbm.at[h].at[pl.ds(w * SPW, SPW)])


@jax.jit
def _sc_call(values, gather_idx, segment_ids):
    mesh = plsc.VectorSubcoreMesh(core_axis_name="c", subcore_axis_name="s")
    kfn = pl.kernel(
        _sc_body,
        mesh=mesh,
        out_type=jax.ShapeDtypeStruct((2, NC * NS * SPW, DH), jnp.float32),
        scratch_types=[
            pltpu.VMEM((CH, DH), jnp.float32),
            pltpu.VMEM((CH, DH), jnp.float32),
            pltpu.VMEM((ACC_ROWS, DH), jnp.float32),
            pltpu.VMEM((ACC_ROWS, L), jnp.float32),
            pltpu.VMEM((CH,), jnp.int32),
            pltpu.VMEM((CH,), jnp.int32),
            pltpu.VMEM((CH,), jnp.int32),
            pltpu.VMEM((CH,), jnp.int32),
            pltpu.VMEM((CH,), jnp.int32),
            pltpu.VMEM((CH,), jnp.int32),
            pltpu.VMEM((CH,), jnp.int32),
            pltpu.VMEM((CH,), jnp.int32),
            pltpu.VMEM((L,), jnp.int32),
            pltpu.SemaphoreType.DMA,
            pltpu.SemaphoreType.DMA,
            pltpu.SemaphoreType.DMA,
            pltpu.SemaphoreType.DMA,
            pltpu.SemaphoreType.DMA,
            pltpu.SemaphoreType.DMA,
        ],
    )
    vh = values.reshape(2 * N_SRC, DH)
    gi = jnp.concatenate([gather_idx, jnp.zeros((PAD,), jnp.int32)])
    si = jnp.concatenate(
        [segment_ids, jnp.full((PAD,), jnp.int32(2**30 - 1), jnp.int32)])
    return kfn(vh, gi, si)


def kernel(values, gather_idx, segment_ids):
    o = _sc_call(values, gather_idx, segment_ids)
    return jnp.concatenate([o[0, :N_SEG], o[1, :N_SEG]], axis=1)
